# Initial kernel scaffold; baseline (speedup 1.0000x reference)
import jax
import jax.numpy as jnp
from jax.experimental import pallas as pl


def kernel(x, edge_index, W):
    raise NotImplementedError("write your pallas kernel here")


# speedup vs baseline: 13.8821x; 13.8821x over previous
import functools

import jax
import jax.numpy as jnp
from jax import lax
from jax.experimental import pallas as pl
from jax.experimental.pallas import tpu as pltpu
from jax.experimental.pallas import tpu_sc as plsc

NC = 2
NS = 16
NW = NC * NS
CHUNK = 128


def _pad_amounts(N, E):
    Np = ((N + 1 + 2047) // 2048) * 2048
    per_tile = -(-E // NW)
    CH = -(-per_tile // CHUNK)
    if CH % 2:
        CH += 1
    Ep = NW * CH * CHUNK
    return Np, Ep, CH


def _sc_deg(col3, ones_h, zeros1, Np, CH):
    mesh = plsc.VectorSubcoreMesh(core_axis_name="c", subcore_axis_name="s")
    rows_per = Np // NS

    @functools.partial(
        pl.kernel,
        out_type=jax.ShapeDtypeStruct((NC, Np), jnp.float32),
        mesh=mesh,
        scratch_types=[
            pltpu.VMEM((CH, CHUNK), jnp.int32),
            pltpu.VMEM((CHUNK,), jnp.float32),
            pltpu.VMEM_SHARED((Np,), jnp.float32),
        ],
    )
    def k(col_h, ones_hbm, z_h, out_h, colv, onesv, acc):
        cid = lax.axis_index("c")
        sid = lax.axis_index("s")
        wid = cid * NS + sid
        r0 = sid * rows_per
        pltpu.sync_copy(z_h.at[pl.ds(r0, rows_per)], acc.at[pl.ds(r0, rows_per)])
        pltpu.sync_copy(ones_hbm, onesv)
        pltpu.sync_copy(col_h.at[wid], colv)
        plsc.subcore_barrier()

        @pl.loop(0, CH)
        def _(j):
            pltpu.sync_copy(onesv, acc.at[colv.at[j]], add=True)

        plsc.subcore_barrier()
        pltpu.sync_copy(acc.at[pl.ds(r0, rows_per)],
                        out_h.at[cid].at[pl.ds(r0, rows_per)])

    return k(col3, ones_h, zeros1)


def _sc_agg(u, row3, col3, zeros2, Np, CH, Cw):
    mesh = plsc.VectorSubcoreMesh(core_axis_name="c", subcore_axis_name="s")
    rows_per = Np // NS

    @functools.partial(
        pl.kernel,
        out_type=jax.ShapeDtypeStruct((NC, Np, Cw), jnp.float32),
        mesh=mesh,
        compiler_params=pltpu.CompilerParams(use_tc_tiling_on_sc=False),
        scratch_types=[
            pltpu.VMEM((CH, CHUNK), jnp.int32),
            pltpu.VMEM((CH, CHUNK), jnp.int32),
            pltpu.VMEM((CHUNK, Cw), jnp.float32),
            pltpu.VMEM_SHARED((Np, Cw), jnp.float32),
            pltpu.SemaphoreType.DMA,
        ],
    )
    def k(u_h, row_h, col_h, z_h, out_h, rowv, colv, buf, acc, sem):
        cid = lax.axis_index("c")
        sid = lax.axis_index("s")
        wid = cid * NS + sid
        r0 = sid * rows_per
        pltpu.sync_copy(z_h.at[pl.ds(r0, rows_per)], acc.at[pl.ds(r0, rows_per)])
        pltpu.sync_copy(row_h.at[wid], rowv)
        pltpu.sync_copy(col_h.at[wid], colv)
        plsc.subcore_barrier()

        @pl.loop(0, CH)
        def _(j):
            pltpu.async_copy(u_h.at[rowv.at[j]], buf, sem).wait()
            pltpu.sync_copy(buf, acc.at[colv.at[j]], add=True)

        plsc.subcore_barrier()
        pltpu.sync_copy(acc.at[pl.ds(r0, rows_per)],
                        out_h.at[cid].at[pl.ds(r0, rows_per)])

    return k(u, row3, col3, zeros2)


def _tc_prep(xp, W, degp):
    Np, D = xp.shape
    C = W.shape[0]

    def body(x_ref, w_ref, deg_ref, u0_ref, dinv_ref, dinv2_ref):
        deg = deg_ref[0] + deg_ref[1]
        pos = deg > 0.0
        dinv = jnp.where(pos, lax.rsqrt(deg), 0.0)
        y = lax.dot_general(x_ref[...], w_ref[...], (((1,), (1,)), ((), ())),
                            preferred_element_type=jnp.float32)
        u0_ref[...] = y * dinv[:, None]
        dinv_ref[...] = dinv
        dinv2_ref[...] = jnp.where(pos, 1.0 / deg, 0.0)

    return pl.pallas_call(
        body,
        out_shape=[
            jax.ShapeDtypeStruct((Np, C), jnp.float32),
            jax.ShapeDtypeStruct((Np,), jnp.float32),
            jax.ShapeDtypeStruct((Np,), jnp.float32),
        ],
    )(xp, W, degp)


def _tc_combine(p, scale):
    _, Np, C = p.shape

    def body(p_ref, s_ref, o_ref):
        o_ref[...] = (p_ref[0] + p_ref[1]) * s_ref[...][:, None]

    return pl.pallas_call(
        body,
        out_shape=jax.ShapeDtypeStruct((Np, C), jnp.float32),
    )(p, scale)


def kernel(x, edge_index, W):
    N, D = x.shape
    C = W.shape[0]
    E = edge_index.shape[1]
    Np, Ep, CH = _pad_amounts(N, E)

    row = edge_index[0]
    col = edge_index[1]
    pad = Ep - E
    row3 = jnp.pad(row, (0, pad), constant_values=N).reshape(NW, CH, CHUNK)
    col3 = jnp.pad(col, (0, pad), constant_values=N).reshape(NW, CH, CHUNK)
    xp = jnp.pad(x, ((0, Np - N), (0, 0)))

    ones_h = jnp.ones((CHUNK,), jnp.float32)
    zeros1 = jnp.zeros((Np,), jnp.float32)
    zeros2 = jnp.zeros((Np, C), jnp.float32)

    degp = _sc_deg(col3, ones_h, zeros1, Np, CH)
    u0, dinv, dinv2 = _tc_prep(xp, W, degp)
    p = _sc_agg(u0, row3, col3, zeros2, Np, CH, C)
    u1 = _tc_combine(p, dinv2)
    q = _sc_agg(u1, row3, col3, zeros2, Np, CH, C)
    outp = _tc_combine(q, dinv)
    return outp[:N]
